# Initial kernel scaffold; baseline (speedup 1.0000x reference)
import jax
import jax.numpy as jnp
from jax.experimental import pallas as pl


def kernel(coordinates, gene_ix, weight1, bias1):
    raise NotImplementedError("write your pallas kernel here")


# speedup vs baseline: 7.5957x; 7.5957x over previous
import functools

import numpy as np
import jax
import jax.numpy as jnp
from jax.experimental import pallas as pl
from jax.experimental.pallas import tpu as pltpu
from jax.experimental.pallas import tpu_sc as plsc

N_FREQ = 10
SINE_DIM = N_FREQ * 2 * 2
D_EMB = 5
ROW = SINE_DIM * D_EMB
ROW_PAD = 208
GATHER_WINDOW = 128
TC_BLOCK = 1024


def _host_constants():
    t = np.arange(2 * N_FREQ)
    freqs = (1.0 / 1000.0 ** (2.0 * (t // 2 + 1) / N_FREQ)).astype(np.float32)
    shifts = np.where(t % 2 == 1, np.pi / 2.0, 0.0).astype(np.float32)

    p = np.arange(ROW_PAD)
    k = p // D_EMB
    fE = np.zeros(ROW_PAD, np.float32)
    sE = np.zeros(ROW_PAD, np.float32)
    fE[:ROW] = freqs[k[:ROW] % (2 * N_FREQ)]
    sE[:ROW] = shifts[k[:ROW] % (2 * N_FREQ)]
    sE[ROW:ROW + D_EMB] = np.pi / 2.0
    S = np.zeros((ROW_PAD, D_EMB), np.float32)
    valid = p < ROW + D_EMB
    S[p[valid], (p % D_EMB)[valid]] = 1.0
    return (jnp.asarray(fE).reshape(1, ROW_PAD),
            jnp.asarray(sE).reshape(1, ROW_PAD),
            jnp.asarray(S))


def _sc_gather(table, idx):
    nfrag = idx.shape[0]
    idx2 = idx.reshape(1, nfrag)
    mesh = plsc.VectorSubcoreMesh(core_axis_name="c", subcore_axis_name="s")

    @functools.partial(
        pl.kernel,
        out_type=jax.ShapeDtypeStruct((nfrag, table.shape[1]), table.dtype),
        mesh=mesh,
        compiler_params=pltpu.CompilerParams(use_tc_tiling_on_sc=False),
    )
    def k(x_hbm, i_hbm, o_hbm):
        def body(i_vmem, o_vmem):
            pltpu.sync_copy(x_hbm.at[i_vmem.at[0]], o_vmem)

        pltpu.emit_pipeline(
            body,
            grid=(nfrag // GATHER_WINDOW,),
            in_specs=[pl.BlockSpec((1, GATHER_WINDOW), lambda i: (0, i))],
            out_specs=[pl.BlockSpec((GATHER_WINDOW, table.shape[1]),
                                    lambda i: (i, 0))],
            core_axis_name=("c", "s"),
            dimension_semantics=(pltpu.PARALLEL,),
        )(i_hbm, o_hbm)

    return k(table, idx2)


def _tc_body(rows_ref, coords_ref, f_ref, s_ref, sel_ref, o_ref):
    c0 = coords_ref[:, 0:1]
    c1 = coords_ref[:, 1:2]
    lane = jax.lax.broadcasted_iota(jnp.int32, (1, ROW_PAD), 1)
    coord = jnp.where(lane < ROW // 2, c0, c1)
    emb = jnp.sin(coord * f_ref[...] + s_ref[...])
    prod = rows_ref[...] * emb
    acc = jax.lax.dot_general(
        prod, sel_ref[...], (((1,), (0,)), ((), ())),
        precision=jax.lax.Precision.HIGHEST,
        preferred_element_type=jnp.float32)
    o_ref[...] = jax.nn.sigmoid(acc)


def _tc_compute(rows, coords, fE, sE, S):
    n = rows.shape[0]
    return pl.pallas_call(
        _tc_body,
        grid=(n // TC_BLOCK,),
        in_specs=[
            pl.BlockSpec((TC_BLOCK, ROW_PAD), lambda i: (i, 0)),
            pl.BlockSpec((TC_BLOCK, 2), lambda i: (i, 0)),
            pl.BlockSpec((1, ROW_PAD), lambda i: (0, 0)),
            pl.BlockSpec((1, ROW_PAD), lambda i: (0, 0)),
            pl.BlockSpec((ROW_PAD, D_EMB), lambda i: (0, 0)),
        ],
        out_specs=pl.BlockSpec((TC_BLOCK, D_EMB), lambda i: (i, 0)),
        out_shape=jax.ShapeDtypeStruct((n, D_EMB), jnp.float32),
    )(rows, coords, fE, sE, S)


def kernel(coordinates, gene_ix, weight1, bias1):
    g = weight1.shape[0]
    table = jnp.concatenate(
        [weight1.reshape(g, ROW), bias1,
         jnp.zeros((g, ROW_PAD - ROW - D_EMB), jnp.float32)], axis=1)
    idx = gene_ix.astype(jnp.int32)
    rows = _sc_gather(table, idx)
    fE, sE, S = _host_constants()
    return _tc_compute(rows, coordinates, fE, sE, S)
